# Initial kernel scaffold; baseline (speedup 1.0000x reference)
import jax
import jax.numpy as jnp
from jax.experimental import pallas as pl


def kernel(input_x, emb_table, W_fc, b_fc):
    raise NotImplementedError("write your pallas kernel here")


# speedup vs baseline: 3.3577x; 3.3577x over previous
import functools

import jax
import jax.numpy as jnp
from jax import lax
from jax.experimental import pallas as pl
from jax.experimental.pallas import tpu as pltpu
from jax.experimental.pallas import tpu_sc as plsc

_NUM_WORKERS = 32
_CHUNK = 128


def _sc_gather(emb_table, idx3d, total, emb):
    nch = (total // _CHUNK) // _NUM_WORKERS
    mesh = plsc.VectorSubcoreMesh(core_axis_name="c", subcore_axis_name="s")

    @functools.partial(
        pl.kernel,
        mesh=mesh,
        out_type=jax.ShapeDtypeStruct((total, emb), jnp.float32),
        compiler_params=pltpu.CompilerParams(use_tc_tiling_on_sc=False),
        scratch_types=[
            pltpu.VMEM((nch, _CHUNK), jnp.int32),
            pltpu.VMEM((_CHUNK, emb), jnp.float32),
            pltpu.SemaphoreType.DMA,
        ],
    )
    def gather_kernel(table_hbm, idx_hbm, out_hbm, idx_v, rows_v, sem):
        wid = lax.axis_index("s") * 2 + lax.axis_index("c")
        chunk0 = wid * nch
        pltpu.sync_copy(idx_hbm.at[wid], idx_v)

        def body(j, carry):
            pltpu.async_copy(table_hbm.at[idx_v.at[j]], rows_v, sem).wait()
            pltpu.sync_copy(
                rows_v, out_hbm.at[pl.ds((chunk0 + j) * _CHUNK, _CHUNK)]
            )
            return carry

        lax.fori_loop(0, nch, body, 0)

    return gather_kernel(emb_table, idx3d)


def _tc_matmul(x, w, b, bt=512):
    bsz, k = x.shape
    n = w.shape[0]

    def body(x_ref, w_ref, b_ref, o_ref):
        o_ref[...] = (
            lax.dot_general(
                x_ref[...], w_ref[...], (((1,), (1,)), ((), ())),
                preferred_element_type=jnp.float32,
            )
            + b_ref[...]
        )

    return pl.pallas_call(
        body,
        grid=(bsz // bt,),
        in_specs=[
            pl.BlockSpec((bt, k), lambda i: (i, 0)),
            pl.BlockSpec((n, k), lambda i: (0, 0)),
            pl.BlockSpec((1, n), lambda i: (0, 0)),
        ],
        out_specs=pl.BlockSpec((bt, n), lambda i: (i, 0)),
        out_shape=jax.ShapeDtypeStruct((bsz, n), jnp.float32),
    )(x, w, b)


def kernel(input_x, emb_table, W_fc, b_fc):
    bsz, nd = input_x.shape
    vocab, emb = emb_table.shape
    out_dim = W_fc.shape[0]
    total = bsz * nd

    nch = (total // _CHUNK) // _NUM_WORKERS
    idx3d = input_x.reshape(_NUM_WORKERS, nch, _CHUNK)
    gathered = _sc_gather(emb_table, idx3d, total, emb)
    x = gathered.reshape(bsz, nd * emb)
    return _tc_matmul(x, W_fc, b_fc.reshape(1, out_dim))


# speedup vs baseline: 3.3581x; 1.0001x over previous
import functools

import jax
import jax.numpy as jnp
from jax import lax
from jax.experimental import pallas as pl
from jax.experimental.pallas import tpu as pltpu
from jax.experimental.pallas import tpu_sc as plsc

_NUM_WORKERS = 32
_CHUNK = 128


def _sc_gather(emb_table, idx3d, total, emb):
    nch = (total // _CHUNK) // _NUM_WORKERS
    mesh = plsc.VectorSubcoreMesh(core_axis_name="c", subcore_axis_name="s")

    @functools.partial(
        pl.kernel,
        mesh=mesh,
        out_type=jax.ShapeDtypeStruct((total, emb), jnp.float32),
        compiler_params=pltpu.CompilerParams(use_tc_tiling_on_sc=False),
        scratch_types=[
            pltpu.VMEM((nch, _CHUNK), jnp.int32),
            pltpu.VMEM((_CHUNK, emb), jnp.float32),
            pltpu.SemaphoreType.DMA,
        ],
    )
    def gather_kernel(table_hbm, idx_hbm, out_hbm, idx_v, rows_v, sem):
        wid = lax.axis_index("s") * 2 + lax.axis_index("c")
        chunk0 = wid * nch
        pltpu.sync_copy(idx_hbm.at[wid], idx_v)

        def body(j, carry):
            pltpu.async_copy(table_hbm.at[idx_v.at[j]], rows_v, sem).wait()
            pltpu.sync_copy(
                rows_v, out_hbm.at[pl.ds((chunk0 + j) * _CHUNK, _CHUNK)]
            )
            return carry

        lax.fori_loop(0, nch, body, 0)

    return gather_kernel(emb_table, idx3d)


def _tc_matmul(x, w, b, bt=512):
    bsz, k = x.shape
    n = w.shape[0]

    def body(x_ref, w_ref, b_ref, o_ref):
        o_ref[...] = (
            lax.dot_general(
                x_ref[...].astype(jnp.bfloat16),
                w_ref[...].astype(jnp.bfloat16),
                (((1,), (1,)), ((), ())),
                preferred_element_type=jnp.float32,
            )
            + b_ref[...]
        )

    return pl.pallas_call(
        body,
        grid=(bsz // bt,),
        in_specs=[
            pl.BlockSpec((bt, k), lambda i: (i, 0)),
            pl.BlockSpec((n, k), lambda i: (0, 0)),
            pl.BlockSpec((1, n), lambda i: (0, 0)),
        ],
        out_specs=pl.BlockSpec((bt, n), lambda i: (i, 0)),
        out_shape=jax.ShapeDtypeStruct((bsz, n), jnp.float32),
    )(x, w, b)


def kernel(input_x, emb_table, W_fc, b_fc):
    bsz, nd = input_x.shape
    vocab, emb = emb_table.shape
    out_dim = W_fc.shape[0]
    total = bsz * nd

    nch = (total // _CHUNK) // _NUM_WORKERS
    idx3d = input_x.reshape(_NUM_WORKERS, nch, _CHUNK)
    gathered = _sc_gather(emb_table, idx3d, total, emb)
    x = gathered.reshape(bsz, nd * emb)
    return _tc_matmul(x, W_fc, b_fc.reshape(1, out_dim))
